# Initial kernel scaffold; baseline (speedup 1.0000x reference)
import jax
import jax.numpy as jnp
from jax.experimental import pallas as pl


def kernel(atom_embedding, bond_embedding, indices_i, indices_j, W1, b1, gamma1, beta1, mean1, var1, W2, b2, gamma2, beta2, mean2, var2):
    raise NotImplementedError("write your pallas kernel here")


# speedup vs baseline: 3.5515x; 3.5515x over previous
import functools

import jax
import jax.numpy as jnp
from jax import lax
from jax.experimental import pallas as pl
from jax.experimental.pallas import tpu as pltpu
from jax.experimental.pallas import tpu_sc as plsc

_N = 10000
_E = 320000
_D = 128
_EPS = 1e-3

_TBLK = 2000


def _tables_body(atom_ref, mi_ref, mj_ref, hbc_ref, pi_ref, pj_ref):
    a = atom_ref[...]
    half_bc = hbc_ref[0:1, :]
    pi_ref[...] = jnp.dot(a, mi_ref[...], preferred_element_type=jnp.float32) + half_bc
    pj_ref[...] = jnp.dot(a, mj_ref[...], preferred_element_type=jnp.float32) + half_bc


def _project_tables(atom, mi, mj, half_bc):
    return pl.pallas_call(
        _tables_body,
        grid=(_N // _TBLK,),
        in_specs=[
            pl.BlockSpec((_TBLK, _D), lambda i: (i, 0)),
            pl.BlockSpec((_D, _D), lambda i: (0, 0)),
            pl.BlockSpec((_D, _D), lambda i: (0, 0)),
            pl.BlockSpec((8, _D), lambda i: (0, 0)),
        ],
        out_specs=[
            pl.BlockSpec((_TBLK, _D), lambda i: (i, 0)),
            pl.BlockSpec((_TBLK, _D), lambda i: (i, 0)),
        ],
        out_shape=[
            jax.ShapeDtypeStruct((_N, _D), jnp.float32),
            jax.ShapeDtypeStruct((_N, _D), jnp.float32),
        ],
    )(atom, mi, mj, half_bc)


_NC = 2
_NS = 16
_NW = _NC * _NS
_PER_W = _E // _NW
_CHUNK = 200
_NCHUNK = _PER_W // _CHUNK


def _make_gather_sum():
    mesh = plsc.VectorSubcoreMesh(core_axis_name="c", subcore_axis_name="s")

    @functools.partial(
        pl.kernel,
        mesh=mesh,
        out_type=jax.ShapeDtypeStruct((_E, _D), jnp.float32),
        scratch_types=[
            pltpu.VMEM((_CHUNK,), jnp.int32),
            pltpu.VMEM((_CHUNK,), jnp.int32),
            pltpu.VMEM((_CHUNK, _D), jnp.float32),
            pltpu.VMEM((_CHUNK, _D), jnp.float32),
            pltpu.SemaphoreType.DMA,
            pltpu.SemaphoreType.DMA,
        ],
    )
    def gather_sum(pi_hbm, pj_hbm, ii_hbm, jj_hbm, out_hbm,
                   idxi_v, idxj_v, bufa, bufb, sema, semb):
        wid = lax.axis_index("s") * _NC + lax.axis_index("c")
        base = wid * _PER_W

        def chunk_body(g, carry):
            row0 = base + g * _CHUNK
            pltpu.sync_copy(ii_hbm.at[pl.ds(row0, _CHUNK)], idxi_v)
            pltpu.sync_copy(jj_hbm.at[pl.ds(row0, _CHUNK)], idxj_v)
            cpa = pltpu.async_copy(pi_hbm.at[idxi_v], bufa, sema)
            cpb = pltpu.async_copy(pj_hbm.at[idxj_v], bufb, semb)
            cpa.wait()
            cpb.wait()

            def add_body(r, c2):
                for l in range(_D // 16):
                    s = pl.ds(l * 16, 16)
                    bufa[r, s] = bufa[r, s] + bufb[r, s]
                return c2

            lax.fori_loop(0, _CHUNK, add_body, 0)
            pltpu.sync_copy(bufa, out_hbm.at[pl.ds(row0, _CHUNK)])
            return carry

        lax.fori_loop(0, _NCHUNK, chunk_body, 0)

    return gather_sum


_EBLK = 4000


def _edge_body(bond_ref, gath_ref, wci_ref, out_ref):
    b = bond_ref[...]
    out_ref[...] = gath_ref[...] + jnp.dot(b, wci_ref[...],
                                           preferred_element_type=jnp.float32)


def _edge_update(bond, gath, wci):
    return pl.pallas_call(
        _edge_body,
        grid=(_E // _EBLK,),
        in_specs=[
            pl.BlockSpec((_EBLK, _D), lambda i: (i, 0)),
            pl.BlockSpec((_EBLK, _D), lambda i: (i, 0)),
            pl.BlockSpec((_D, _D), lambda i: (0, 0)),
        ],
        out_specs=pl.BlockSpec((_EBLK, _D), lambda i: (i, 0)),
        out_shape=jax.ShapeDtypeStruct((_E, _D), jnp.float32),
    )(bond, gath, wci)


def kernel(atom_embedding, bond_embedding, indices_i, indices_j,
           W1, b1, gamma1, beta1, mean1, var1,
           W2, b2, gamma2, beta2, mean2, var2):
    s1 = gamma1 / jnp.sqrt(var1 + _EPS)
    t1 = beta1 - mean1 * s1
    s2 = gamma2 / jnp.sqrt(var2 + _EPS)
    t2 = beta2 - mean2 * s2
    wf = (s1[:, None] * W2) * s2[None, :]
    bf = (t1 @ W2 + b2) * s2 + t2
    mi = W1[:_D] @ wf
    wc = W1[_D:2 * _D] @ wf
    mj = W1[2 * _D:] @ wf
    bc = b1 @ wf + bf
    wci = wc + jnp.eye(_D, dtype=jnp.float32)
    half_bc = jnp.broadcast_to(0.5 * bc, (8, _D))

    pi, pj = _project_tables(atom_embedding, mi, mj, half_bc)
    gath = _make_gather_sum()(pi, pj, indices_i, indices_j)
    return _edge_update(bond_embedding, gath, wci)
